# baseline (speedup 1.0000x reference)
import functools
import math

import jax
import jax.numpy as jnp
from jax import lax
from jax.experimental import pallas as pl
from jax.experimental.pallas import tpu as pltpu
from jax.experimental.pallas import tpu_sc as plsc

_VOCAB = 1000000
_D = 64
_MAX_LEN = 50
_BATCH = 4096

_NC = 2
_NS = 16
_NW = _NC * _NS
_BPW = _BATCH // _NW
_SCALE = math.sqrt(_D)
_LANES = 16


def _pe_table():
    pos = jnp.arange(_MAX_LEN, dtype=jnp.float32)[:, None]
    i_even = jnp.arange(0, _D, 2, dtype=jnp.float32)[None, :]
    sin_part = jnp.sin(pos / jnp.power(10000.0, 2.0 * i_even / _D))
    cos_part = jnp.cos(pos / jnp.power(10000.0, 2.0 * (i_even + 1.0) / _D))
    pe = jnp.zeros((_MAX_LEN, _D), dtype=jnp.float32)
    pe = pe.at[:, 0::2].set(sin_part)
    pe = pe.at[:, 1::2].set(cos_part)
    return pe


_mesh = plsc.VectorSubcoreMesh(core_axis_name="c", subcore_axis_name="s")


@functools.partial(
    pl.kernel,
    mesh=_mesh,
    out_type=jax.ShapeDtypeStruct(
        (_MAX_LEN, _D // 8, _BATCH // 128, 8, 128), jnp.float32),
    compiler_params=pltpu.CompilerParams(
        use_tc_tiling_on_sc=False, needs_layout_passes=False),
    scratch_types=[
        pltpu.VMEM((_MAX_LEN, _BPW), jnp.int32),
        pltpu.VMEM((_MAX_LEN, _D), jnp.float32),
        pltpu.VMEM((_BPW, _D), jnp.float32),
        pltpu.VMEM((_BPW, _D), jnp.float32),
        pltpu.VMEM((8, 8, _BPW), jnp.float32),
        pltpu.VMEM((8, 8, _BPW), jnp.float32),
        pltpu.SemaphoreType.DMA,
        pltpu.SemaphoreType.DMA,
        pltpu.SemaphoreType.DMA,
        pltpu.SemaphoreType.DMA,
    ],
)
def _sc_embed(idx_hbm, pe_hbm, table_hbm, out_hbm,
              idx_v, pe_v, buf0, buf1, tb0, tb1, g0, g1, w0, w1):
    wid = lax.axis_index("s") * _NC + lax.axis_index("c")
    bs = wid * _BPW

    pltpu.sync_copy(idx_hbm.at[:, pl.ds(bs, _BPW)], idx_v)
    pltpu.sync_copy(pe_hbm, pe_v)

    def gather_start(t, buf, sem):
        pltpu.async_copy(table_hbm.at[idx_v.at[t]], buf, sem)

    def gather_wait(buf, sem):
        pltpu.make_async_copy(table_hbm.at[idx_v.at[0]], buf, sem).wait()

    def wb_start(t, tb, sem):
        pltpu.async_copy(tb, out_hbm.at[t, :, wid], sem)

    def wb_wait(tb, sem):
        pltpu.make_async_copy(tb, out_hbm.at[0, :, wid], sem).wait()

    lane = lax.broadcasted_iota(jnp.int32, (_LANES,), 0)
    dtvs = [lane // 8 + 2 * dv for dv in range(_D // _LANES)]
    ddv = lane % 8

    def compute(t, buf, tb):
        pevs = [pe_v[t, pl.ds(dv * _LANES, _LANES)]
                for dv in range(_D // _LANES)]

        def jbody(j, c):
            jv = jnp.zeros((_LANES,), jnp.int32) + j
            for dv in range(_D // _LANES):
                vals = buf[j, pl.ds(dv * _LANES, _LANES)]
                plsc.store_scatter(
                    tb, [dtvs[dv], ddv, jv], vals * _SCALE + pevs[dv])
            return c

        lax.fori_loop(0, _BPW, jbody, 0, unroll=4)

    gather_start(0, buf0, g0)

    def body(h, c):
        t0 = 2 * h
        t1 = 2 * h + 1

        @pl.when(h > 0)
        def _():
            wb_wait(tb1, w1)

        gather_start(t1, buf1, g1)
        gather_wait(buf0, g0)
        compute(t0, buf0, tb0)
        wb_start(t0, tb0, w0)

        @pl.when(h + 1 < _MAX_LEN // 2)
        def _():
            wb_wait(tb0, w0)
            gather_start(t0 + 2, buf0, g0)

        gather_wait(buf1, g1)
        compute(t1, buf1, tb1)
        wb_start(t1, tb1, w1)
        return c

    lax.fori_loop(0, _MAX_LEN // 2, body, 0)

    wb_wait(tb0, w0)
    wb_wait(tb1, w1)


def kernel(encoded_words, embed_weight):
    idx_t = encoded_words.astype(jnp.int32).T
    pe = _pe_table()
    out5 = _sc_embed(idx_t, pe, embed_weight)
    out = out5.transpose(2, 4, 0, 1, 3).reshape(_BATCH, _MAX_LEN, _D)
    return out


# speedup vs baseline: 1.2052x; 1.2052x over previous
import functools
import math

import jax
import jax.numpy as jnp
from jax import lax
from jax.experimental import pallas as pl
from jax.experimental.pallas import tpu as pltpu
from jax.experimental.pallas import tpu_sc as plsc

_VOCAB = 1000000
_D = 64
_MAX_LEN = 50
_BATCH = 4096

_NC = 2
_NS = 16
_NW = _NC * _NS
_BPW = _BATCH // _NW
_SCALE = math.sqrt(_D)
_LANES = 16


def _pe_table():
    pos = jnp.arange(_MAX_LEN, dtype=jnp.float32)[:, None]
    i_even = jnp.arange(0, _D, 2, dtype=jnp.float32)[None, :]
    sin_part = jnp.sin(pos / jnp.power(10000.0, 2.0 * i_even / _D))
    cos_part = jnp.cos(pos / jnp.power(10000.0, 2.0 * (i_even + 1.0) / _D))
    pe = jnp.zeros((_MAX_LEN, _D), dtype=jnp.float32)
    pe = pe.at[:, 0::2].set(sin_part)
    pe = pe.at[:, 1::2].set(cos_part)
    return pe


_mesh = plsc.VectorSubcoreMesh(core_axis_name="c", subcore_axis_name="s")


@functools.partial(
    pl.kernel,
    mesh=_mesh,
    out_type=jax.ShapeDtypeStruct(
        (_MAX_LEN, _D // 8, _BATCH // 128, 8, 128), jnp.float32),
    compiler_params=pltpu.CompilerParams(
        use_tc_tiling_on_sc=False, needs_layout_passes=False),
    scratch_types=[
        pltpu.VMEM((_MAX_LEN, _BPW), jnp.int32),
        pltpu.VMEM((_MAX_LEN, _D), jnp.float32),
        pltpu.VMEM((_BPW, _D), jnp.float32),
        pltpu.VMEM((_BPW, _D), jnp.float32),
        pltpu.VMEM((8, 8, _BPW + 1), jnp.float32),
        pltpu.VMEM((8, 8, _BPW + 1), jnp.float32),
        pltpu.SemaphoreType.DMA,
        pltpu.SemaphoreType.DMA,
        pltpu.SemaphoreType.DMA,
        pltpu.SemaphoreType.DMA,
    ],
)
def _sc_embed(idx_hbm, pe_hbm, table_hbm, out_hbm,
              idx_v, pe_v, buf0, buf1, tb0, tb1, g0, g1, w0, w1):
    wid = lax.axis_index("s") * _NC + lax.axis_index("c")
    bs = wid * _BPW

    pltpu.sync_copy(idx_hbm.at[:, pl.ds(bs, _BPW)], idx_v)
    pltpu.sync_copy(pe_hbm, pe_v)

    def gather_start(t, buf, sem):
        pltpu.async_copy(table_hbm.at[idx_v.at[t]], buf, sem)

    def gather_wait(buf, sem):
        pltpu.make_async_copy(table_hbm.at[idx_v.at[0]], buf, sem).wait()

    def wb_start(t, tb, sem):
        pltpu.async_copy(
            tb.at[:, :, pl.ds(0, _BPW)], out_hbm.at[t, :, wid], sem)

    def wb_wait(tb, sem):
        pltpu.make_async_copy(
            tb.at[:, :, pl.ds(0, _BPW)], out_hbm.at[0, :, wid], sem).wait()

    lane = lax.broadcasted_iota(jnp.int32, (_LANES,), 0)
    dtvs = [lane // 8 + 2 * dv for dv in range(_D // _LANES)]
    ddv = lane % 8

    def compute(t, buf, tb):
        pevs = [pe_v[t, pl.ds(dv * _LANES, _LANES)]
                for dv in range(_D // _LANES)]

        def jbody(j, c):
            jv = jnp.zeros((_LANES,), jnp.int32) + j
            for dv in range(_D // _LANES):
                vals = buf[j, pl.ds(dv * _LANES, _LANES)]
                plsc.store_scatter(
                    tb, [dtvs[dv], ddv, jv], vals * _SCALE + pevs[dv])
            return c

        lax.fori_loop(0, _BPW, jbody, 0, unroll=4)

    gather_start(0, buf0, g0)

    def body(h, c):
        t0 = 2 * h
        t1 = 2 * h + 1

        @pl.when(h > 0)
        def _():
            wb_wait(tb1, w1)

        gather_start(t1, buf1, g1)
        gather_wait(buf0, g0)
        compute(t0, buf0, tb0)
        wb_start(t0, tb0, w0)

        @pl.when(h + 1 < _MAX_LEN // 2)
        def _():
            wb_wait(tb0, w0)
            gather_start(t0 + 2, buf0, g0)

        gather_wait(buf1, g1)
        compute(t1, buf1, tb1)
        wb_start(t1, tb1, w1)
        return c

    lax.fori_loop(0, _MAX_LEN // 2, body, 0)

    wb_wait(tb0, w0)
    wb_wait(tb1, w1)


def kernel(encoded_words, embed_weight):
    idx_t = encoded_words.astype(jnp.int32).T
    pe = _pe_table()
    out5 = _sc_embed(idx_t, pe, embed_weight)
    out = out5.transpose(2, 4, 0, 1, 3).reshape(_BATCH, _MAX_LEN, _D)
    return out
